# baseline (speedup 1.0000x reference)
import functools

import jax
import jax.numpy as jnp
from jax import lax
from jax.experimental import pallas as pl
from jax.experimental.pallas import tpu as pltpu
from jax.experimental.pallas import tpu_sc as plsc

D_MODEL = 2048
LANES = 16
NUM_WORKERS = 32
K = 8
NS = 2
NO = 4


@jax.jit
def _gather_add(seq_tok, struct_tok, seq_table, struct_table):
    n = seq_tok.shape[0]
    per_w = n // NUM_WORKERS
    n_chunks = per_w // K
    n_outer = n_chunks // NO
    mesh = plsc.VectorSubcoreMesh(core_axis_name="c", subcore_axis_name="s")

    @functools.partial(
        pl.kernel,
        mesh=mesh,
        out_type=jax.ShapeDtypeStruct((n, D_MODEL), jnp.float32),
        scratch_types=[
            pltpu.VMEM((per_w,), jnp.int32),
            pltpu.VMEM((per_w,), jnp.int32),
            pltpu.VMEM((NS, K, D_MODEL), jnp.float32),
            pltpu.VMEM((NO, K, D_MODEL), jnp.float32),
            pltpu.SemaphoreType.DMA,
            pltpu.SemaphoreType.DMA,
            pltpu.SemaphoreType.DMA,
            pltpu.SemaphoreType.DMA,
            pltpu.SemaphoreType.DMA,
            pltpu.SemaphoreType.DMA,
            pltpu.SemaphoreType.DMA,
            pltpu.SemaphoreType.DMA,
            pltpu.SemaphoreType.DMA,
            pltpu.SemaphoreType.DMA,
        ],
    )
    def k(seq_tok_hbm, struct_tok_hbm, seq_tab_hbm, struct_tab_hbm, out_hbm,
          sidx, tidx, buf_s, buf_o,
          sem_s0, sem_s1, sem_o0, sem_o1, sem_o2, sem_o3,
          sem_w0, sem_w1, sem_w2, sem_w3):
        sem_s = (sem_s0, sem_s1)
        sem_o = (sem_o0, sem_o1, sem_o2, sem_o3)
        sem_w = (sem_w0, sem_w1, sem_w2, sem_w3)
        wid = lax.axis_index("s") * 2 + lax.axis_index("c")
        base = wid * per_w
        pltpu.sync_copy(seq_tok_hbm.at[pl.ds(base, per_w)], sidx)
        pltpu.sync_copy(struct_tok_hbm.at[pl.ds(base, per_w)], tidx)

        def gather_pair(off, bs, bo):
            pltpu.async_copy(
                seq_tab_hbm.at[sidx.at[pl.ds(off, K)]], buf_s.at[bs], sem_s[bs])
            pltpu.async_copy(
                struct_tab_hbm.at[tidx.at[pl.ds(off, K)]], buf_o.at[bo],
                sem_o[bo])

        def wait_gather_pair(off, bs, bo):
            pltpu.make_async_copy(
                seq_tab_hbm.at[sidx.at[pl.ds(off, K)]], buf_s.at[bs],
                sem_s[bs]).wait()
            pltpu.make_async_copy(
                struct_tab_hbm.at[tidx.at[pl.ds(off, K)]], buf_o.at[bo],
                sem_o[bo]).wait()

        def wait_writeback(off, bo):
            pltpu.make_async_copy(
                buf_o.at[bo], out_hbm.at[pl.ds(base + off, K)], sem_w[bo]).wait()

        for g in range(2):
            gather_pair(g * K, g % NS, g % NO)

        def outer(o, _):
            for b in range(NO):
                g = o * NO + b
                off = g * K
                bs = b % NS
                wait_gather_pair(off, bs, b)
                for i in range(K):
                    @plsc.parallel_loop(0, D_MODEL, LANES, unroll=8)
                    def _(j, bs=bs, b=b, i=i):
                        sl = pl.ds(j, LANES)
                        plsc.addupdate(buf_o.at[b, i, sl], buf_s[bs, i, sl])
                pltpu.async_copy(
                    buf_o.at[b], out_hbm.at[pl.ds(base + off, K)], sem_w[b])
                bo2 = (b + 2) % NO
                if b < 2:
                    @pl.when(o > 0)
                    def _():
                        wait_writeback(off - 2 * K, bo2)
                    gather_pair(off + 2 * K, bs, bo2)
                else:
                    @pl.when(o < n_outer - 1)
                    def _():
                        wait_writeback(off - 2 * K, bo2)
                        gather_pair(off + 2 * K, bs, bo2)
            return 0

        lax.fori_loop(0, n_outer, outer, 0)

        for g in range(n_chunks - NO, n_chunks):
            wait_writeback(g * K, g % NO)

    return k(seq_tok, struct_tok, seq_table, struct_table)


def kernel(sequence_tokens, structure_tokens, seq_table, struct_table):
    b, s = sequence_tokens.shape
    n = b * s
    seq_tok = sequence_tokens.reshape(n).astype(jnp.int32)
    struct_tok = structure_tokens.reshape(n).astype(jnp.int32)
    out = _gather_add(seq_tok, struct_tok, seq_table, struct_table)
    return out.reshape(b, s, D_MODEL)


# speedup vs baseline: 1.2535x; 1.2535x over previous
import functools

import jax
import jax.numpy as jnp
from jax import lax
from jax.experimental import pallas as pl
from jax.experimental.pallas import tpu as pltpu
from jax.experimental.pallas import tpu_sc as plsc

D_MODEL = 2048
LANES = 16
N_TOKEN_BLOCKS = 16
N_COL_HALVES = 2
COLS = D_MODEL // N_COL_HALVES
K = 8
NO = 4


@jax.jit
def _gather_add(seq_tok, struct_tok, seq_table, struct_table):
    n = seq_tok.shape[0]
    v_seq = seq_table.shape[0]
    per_w = n // N_TOKEN_BLOCKS
    n_chunks = per_w // K
    n_outer = n_chunks // NO
    mesh = plsc.VectorSubcoreMesh(core_axis_name="c", subcore_axis_name="s")

    @functools.partial(
        pl.kernel,
        mesh=mesh,
        compiler_params=pltpu.CompilerParams(needs_layout_passes=False),
        out_type=jax.ShapeDtypeStruct((n, D_MODEL), jnp.float32),
        scratch_types=[
            pltpu.VMEM((per_w + LANES,), jnp.int32),
            pltpu.VMEM((per_w,), jnp.int32),
            pltpu.VMEM((v_seq, COLS), jnp.float32),
            pltpu.VMEM((NO, K, COLS), jnp.float32),
            pltpu.SemaphoreType.DMA,
            pltpu.SemaphoreType.DMA,
            pltpu.SemaphoreType.DMA,
            pltpu.SemaphoreType.DMA,
            pltpu.SemaphoreType.DMA,
            pltpu.SemaphoreType.DMA,
            pltpu.SemaphoreType.DMA,
            pltpu.SemaphoreType.DMA,
        ],
    )
    def k(seq_tok_hbm, struct_tok_hbm, seq_tab_hbm, struct_tab_hbm, out_hbm,
          sidx, tidx, seq_vmem, buf_o,
          sem_o0, sem_o1, sem_o2, sem_o3,
          sem_w0, sem_w1, sem_w2, sem_w3):
        sem_o = (sem_o0, sem_o1, sem_o2, sem_o3)
        sem_w = (sem_w0, sem_w1, sem_w2, sem_w3)
        tb = lax.axis_index("s")
        ch = lax.axis_index("c")
        base = tb * per_w
        col0 = ch * COLS
        pltpu.sync_copy(seq_tok_hbm.at[pl.ds(base, per_w)],
                        sidx.at[pl.ds(0, per_w)])
        pltpu.sync_copy(struct_tok_hbm.at[pl.ds(base, per_w)], tidx)
        pltpu.sync_copy(seq_tab_hbm.at[:, pl.ds(col0, COLS)], seq_vmem)

        def gather(off, bo):
            pltpu.async_copy(
                struct_tab_hbm.at[tidx.at[pl.ds(off, K)], pl.ds(col0, COLS)],
                buf_o.at[bo], sem_o[bo])

        def wait_gather(off, bo):
            pltpu.make_async_copy(
                struct_tab_hbm.at[tidx.at[pl.ds(off, K)], pl.ds(col0, COLS)],
                buf_o.at[bo], sem_o[bo]).wait()

        def writeback(off, bo):
            return pltpu.make_async_copy(
                buf_o.at[bo],
                out_hbm.at[pl.ds(base + off, K), pl.ds(col0, COLS)],
                sem_w[bo])

        for g in range(2):
            gather(g * K, g % NO)

        lanes_iota = lax.iota(jnp.int32, LANES)

        def outer(o, _):
            for b in range(NO):
                g = o * NO + b
                off = g * K
                wait_gather(off, b)
                rows16 = sidx[pl.ds(off, LANES)]
                for i in range(K):
                    splat = jnp.take_along_axis(
                        rows16, jnp.full((LANES,), i, jnp.int32), axis=0)

                    @plsc.parallel_loop(0, COLS, LANES, unroll=8)
                    def _(j, b=b, i=i, splat=splat):
                        vals = plsc.load_gather(
                            seq_vmem, [splat, lanes_iota + j])
                        plsc.addupdate(buf_o.at[b, i, pl.ds(j, LANES)], vals)
                writeback(off, b).start()
                bo2 = (b + 2) % NO
                if b < 2:
                    @pl.when(o > 0)
                    def _():
                        writeback(off - 2 * K, bo2).wait()
                    gather(off + 2 * K, bo2)
                else:
                    @pl.when(o < n_outer - 1)
                    def _():
                        writeback(off - 2 * K, bo2).wait()
                        gather(off + 2 * K, bo2)
            return 0

        lax.fori_loop(0, n_outer, outer, 0)

        for g in range(n_chunks - NO, n_chunks):
            writeback(g * K, g % NO).wait()

    return k(seq_tok, struct_tok, seq_table, struct_table)


def kernel(sequence_tokens, structure_tokens, seq_table, struct_table):
    b, s = sequence_tokens.shape
    n = b * s
    seq_tok = sequence_tokens.reshape(n).astype(jnp.int32)
    struct_tok = structure_tokens.reshape(n).astype(jnp.int32)
    out = _gather_add(seq_tok, struct_tok, seq_table, struct_table)
    return out.reshape(b, s, D_MODEL)


# speedup vs baseline: 1.6117x; 1.2858x over previous
import functools

import jax
import jax.numpy as jnp
from jax import lax
from jax.experimental import pallas as pl
from jax.experimental.pallas import tpu as pltpu
from jax.experimental.pallas import tpu_sc as plsc

D_MODEL = 2048
LANES = 16
NUM_WORKERS = 32
K = 8
NO = 3


def _pack_seq_table(seq_table):
    bf = seq_table.astype(jnp.bfloat16).reshape(seq_table.shape[0], -1, 2, LANES)
    bits = lax.bitcast_convert_type(bf, jnp.uint16).astype(jnp.uint32)
    words = bits[:, :, 0, :] | (bits[:, :, 1, :] << 16)
    return lax.bitcast_convert_type(words, jnp.int32).reshape(
        seq_table.shape[0], seq_table.shape[1] // 2)


@jax.jit
def _gather_add(seq_tok, struct_tok, seq_packed, struct_table):
    n = seq_tok.shape[0]
    v_seq = seq_packed.shape[0]
    per_w = n // NUM_WORKERS
    n_chunks = per_w // K
    n_loop = n_chunks - 1
    n_outer = n_loop // NO
    mesh = plsc.VectorSubcoreMesh(core_axis_name="c", subcore_axis_name="s")

    @functools.partial(
        pl.kernel,
        mesh=mesh,
        compiler_params=pltpu.CompilerParams(needs_layout_passes=False),
        out_type=jax.ShapeDtypeStruct((n, D_MODEL), jnp.float32),
        scratch_types=[
            pltpu.VMEM((per_w + LANES,), jnp.int32),
            pltpu.VMEM((per_w,), jnp.int32),
            pltpu.VMEM((v_seq, D_MODEL // 2), jnp.int32),
            pltpu.VMEM((NO, K, D_MODEL), jnp.float32),
            pltpu.SemaphoreType.DMA,
            pltpu.SemaphoreType.DMA,
            pltpu.SemaphoreType.DMA,
            pltpu.SemaphoreType.DMA,
            pltpu.SemaphoreType.DMA,
            pltpu.SemaphoreType.DMA,
        ],
    )
    def k(seq_tok_hbm, struct_tok_hbm, seq_pk_hbm, struct_tab_hbm, out_hbm,
          sidx, tidx, seq_pk, buf_o,
          sem_o0, sem_o1, sem_o2, sem_w0, sem_w1, sem_w2):
        sem_o = (sem_o0, sem_o1, sem_o2)
        sem_w = (sem_w0, sem_w1, sem_w2)
        wid = lax.axis_index("s") * 2 + lax.axis_index("c")
        base = wid * per_w
        pltpu.sync_copy(seq_tok_hbm.at[pl.ds(base, per_w)],
                        sidx.at[pl.ds(0, per_w)])
        pltpu.sync_copy(struct_tok_hbm.at[pl.ds(base, per_w)], tidx)
        pltpu.sync_copy(seq_pk_hbm, seq_pk)

        def gather(off, bo):
            pltpu.async_copy(
                struct_tab_hbm.at[tidx.at[pl.ds(off, K)]], buf_o.at[bo],
                sem_o[bo])

        def wait_gather(off, bo):
            pltpu.make_async_copy(
                struct_tab_hbm.at[tidx.at[pl.ds(off, K)]], buf_o.at[bo],
                sem_o[bo]).wait()

        def writeback(off, bo):
            return pltpu.make_async_copy(
                buf_o.at[bo], out_hbm.at[pl.ds(base + off, K)], sem_w[bo])

        lanes_iota = lax.iota(jnp.int32, LANES)
        himask = jnp.full((LANES,), -65536, jnp.int32)

        def do_chunk(g_off, b):
            rows16 = sidx[pl.ds(g_off, LANES)]
            for i in range(K):
                splat = jnp.take_along_axis(
                    rows16, jnp.full((LANES,), i, jnp.int32), axis=0)

                @plsc.parallel_loop(0, D_MODEL, 2 * LANES, unroll=4)
                def _(j, b=b, i=i, splat=splat):
                    w = plsc.load_gather(
                        seq_pk, [splat, lanes_iota + lax.shift_right_logical(j, 1)])
                    lo = plsc.bitcast(lax.shift_left(w, 16), jnp.float32)
                    hi = plsc.bitcast(w & himask, jnp.float32)
                    plsc.addupdate(buf_o.at[b, i, pl.ds(j, LANES)], lo)
                    plsc.addupdate(buf_o.at[b, i, pl.ds(j + LANES, LANES)], hi)

        gather(0, 0)
        gather(K, 1)

        def outer(o, _):
            for b in range(NO):
                off = (o * NO + b) * K
                wait_gather(off, b)
                do_chunk(off, b)
                writeback(off, b).start()
                bo2 = (b + 2) % NO
                if b == 0:
                    @pl.when(o > 0)
                    def _():
                        writeback(off - K, bo2).wait()
                    gather(off + 2 * K, bo2)
                elif b == 1:
                    writeback(off - K, bo2).wait()
                    gather(off + 2 * K, bo2)
                else:
                    @pl.when(o < n_outer - 1)
                    def _():
                        writeback(off - K, bo2).wait()
                        gather(off + 2 * K, bo2)
            return 0

        lax.fori_loop(0, n_outer, outer, 0)

        last = n_loop * K
        wait_gather(last, 0)
        do_chunk(last, 0)
        writeback(last, 0).start()
        writeback(last - 2 * K, 1).wait()
        writeback(last - K, 2).wait()
        writeback(last, 0).wait()

    return k(seq_tok, struct_tok, seq_packed, struct_table)


def kernel(sequence_tokens, structure_tokens, seq_table, struct_table):
    b, s = sequence_tokens.shape
    n = b * s
    seq_tok = sequence_tokens.reshape(n).astype(jnp.int32)
    struct_tok = structure_tokens.reshape(n).astype(jnp.int32)
    out = _gather_add(seq_tok, struct_tok, _pack_seq_table(seq_table),
                      struct_table)
    return out.reshape(b, s, D_MODEL)
